# baseline (speedup 1.0000x reference)
import functools

import jax
import jax.numpy as jnp
from jax import lax
from jax.experimental import pallas as pl
from jax.experimental.pallas import tpu as pltpu
from jax.experimental.pallas import tpu_sc as plsc

_N = 100000
_H = 32
_B = 1024
_SS = 20

_NC = 2
_NS = 16
_NW = _NC * _NS
_BT = 2 * _B + 2 * _B * _SS
_SPP = _BT // (_NW // 2)
_L = 16

_mesh = plsc.VectorSubcoreMesh(core_axis_name="c", subcore_axis_name="s")


def _dots_body(w_ref, embt_ref, s0_ref, s1_ref):
    W = w_ref[...]
    wsym = 0.5 * (W[:, :_H] + W[:, _H:])
    S2 = lax.dot_general(
        wsym, embt_ref[...], (((1,), (0,)), ((), ())),
        preferred_element_type=jnp.float32,
    )
    s0_ref[...] = S2[0]
    s1_ref[...] = S2[1]


_dots_tc = pl.pallas_call(
    _dots_body,
    out_shape=(
        jax.ShapeDtypeStruct((_N,), jnp.float32),
        jax.ShapeDtypeStruct((_N,), jnp.float32),
    ),
    in_specs=[
        pl.BlockSpec(memory_space=pltpu.VMEM),
        pl.BlockSpec(memory_space=pltpu.VMEM),
    ],
)


_NQ = _N // 4
_NQP = _NQ + _L
_SPT = _BT // 4


@functools.partial(
    pl.kernel,
    mesh=_mesh,
    out_type=tuple(
        jax.ShapeDtypeStruct((_BT,), jnp.float32) for _ in range(8)
    ),
    scratch_types=[
        pltpu.VMEM((_NQP,), jnp.float32),
        pltpu.VMEM((_SPT,), jnp.int32),
        pltpu.VMEM((_SPT,), jnp.float32),
    ],
    compiler_params=pltpu.CompilerParams(
        use_tc_tiling_on_sc=False, needs_layout_passes=False
    ),
)
def _gather_sc(s0_hbm, s1_hbm, idx_hbm, g00, g01, g02, g03, g10, g11, g12,
               g13, s_v, idx_v, out_v):
    wid = lax.axis_index("s") * _NC + lax.axis_index("c")
    kk = wid % 2
    quarter = (wid // 2) % 4
    base = (wid // 8) * _SPT
    pltpu.sync_copy(idx_hbm.at[quarter, pl.ds(base, _SPT)], idx_v)

    @pl.when(kk == 0)
    def _():
        pltpu.sync_copy(s0_hbm.at[pl.ds(quarter * _NQ, _NQ)],
                        s_v.at[pl.ds(0, _NQ)])

    @pl.when(kk == 1)
    def _():
        pltpu.sync_copy(s1_hbm.at[pl.ds(quarter * _NQ, _NQ)],
                        s_v.at[pl.ds(0, _NQ)])

    s_v[pl.ds(_NQ, _L)] = jnp.zeros((_L,), jnp.float32)

    for c in range(_SPT // _L):
        iv = idx_v[pl.ds(c * _L, _L)]
        out_v[pl.ds(c * _L, _L)] = plsc.load_gather(s_v, [iv])

    outs = [[g00, g01, g02, g03], [g10, g11, g12, g13]]
    for kx in (0, 1):
        for qx in range(4):
            @pl.when((kk == kx) & (quarter == qx))
            def _(o=outs[kx][qx]):
                pltpu.sync_copy(out_v, o.at[pl.ds(base, _SPT)])


def _softplus(g, p):
    r = jnp.clip(g / p, -75.0, 75.0)
    return p * jnp.log1p(jnp.exp(r))


def _final_body(b_ref, psi_ref, k_ref, g00_ref, g01_ref, g02_ref, g03_ref,
                g10_ref, g11_ref, g12_ref, g13_ref, lam_ref, ls_ref):
    s0 = (g00_ref[...] + g01_ref[...]) + (g02_ref[...] + g03_ref[...])
    s1 = (g10_ref[...] + g11_ref[...]) + (g12_ref[...] + g13_ref[...])
    b0 = b_ref[0]
    b1 = b_ref[1]
    p0 = psi_ref[0]
    p1 = psi_ref[1]

    su0 = lax.slice(s0, (0,), (_B,))
    su1 = lax.slice(s1, (0,), (_B,))
    sv0 = lax.slice(s0, (_B,), (2 * _B,))
    sv1 = lax.slice(s1, (_B,), (2 * _B,))

    kk = k_ref[...]
    lam0 = _softplus(su0 + sv0 + b0, p0)
    lam1 = _softplus(su1 + sv1 + b1, p1)
    lam_ref[...] = jnp.where(kk == 0, lam0, lam1)

    ovo = 2 * _B
    ouo = ovo + _SS * _B
    for s in range(_SS):
        cv = ovo + s * _B
        cu = ouo + s * _B
        acc = (
            _softplus(su0 + lax.slice(s0, (cv,), (cv + _B,)) + b0, p0)
            + _softplus(su1 + lax.slice(s1, (cv,), (cv + _B,)) + b1, p1)
            + _softplus(sv0 + lax.slice(s0, (cu,), (cu + _B,)) + b0, p0)
            + _softplus(sv1 + lax.slice(s1, (cu,), (cu + _B,)) + b1, p1)
        )
        ls_ref[0, s] = jnp.sum(acc) * (1.0 / _SS)


_final_tc = pl.pallas_call(
    _final_body,
    out_shape=(
        jax.ShapeDtypeStruct((_B,), jnp.float32),
        jax.ShapeDtypeStruct((1, _SS), jnp.float32),
    ),
    in_specs=[pl.BlockSpec(memory_space=pltpu.SMEM)] * 2
    + [pl.BlockSpec(memory_space=pltpu.VMEM)] * 9,
    out_specs=(
        pl.BlockSpec(memory_space=pltpu.VMEM),
        pl.BlockSpec(memory_space=pltpu.SMEM),
    ),
)


def kernel(embeddings, W_omega, b_omega, psi, t, u, v, k, u_others, v_others):
    del t
    s0, s1 = _dots_tc(W_omega, embeddings.T)

    idx = jnp.concatenate([
        u.astype(jnp.int32),
        v.astype(jnp.int32),
        v_others.astype(jnp.int32).T.reshape(-1),
        u_others.astype(jnp.int32).T.reshape(-1),
    ])
    q = idx // _NQ
    local = idx - q * _NQ
    idx4 = jnp.where(q[None, :] == jnp.arange(4, dtype=jnp.int32)[:, None],
                     local[None, :], _NQ)
    gs = _gather_sc(s0, s1, idx4)
    lam, ls = _final_tc(b_omega, psi, k.astype(jnp.int32), *gs)
    return (lam, ls.reshape(_SS))


# speedup vs baseline: 1.0395x; 1.0395x over previous
import functools

import jax
import jax.numpy as jnp
from jax import lax
from jax.experimental import pallas as pl
from jax.experimental.pallas import tpu as pltpu
from jax.experimental.pallas import tpu_sc as plsc

_N = 100000
_H = 32
_B = 1024
_SS = 20

_NC = 2
_NS = 16
_NW = _NC * _NS
_BT = 2 * _B + 2 * _B * _SS
_SPP = _BT // (_NW // 2)
_L = 16

_mesh = plsc.VectorSubcoreMesh(core_axis_name="c", subcore_axis_name="s")


def _dots_body(w_ref, embt_ref, s0_ref, s1_ref):
    W = w_ref[...]
    wsym = 0.5 * (W[:, :_H] + W[:, _H:])
    S2 = lax.dot_general(
        wsym, embt_ref[...], (((1,), (0,)), ((), ())),
        preferred_element_type=jnp.float32,
    )
    s0_ref[...] = S2[0]
    s1_ref[...] = S2[1]


_dots_tc = pl.pallas_call(
    _dots_body,
    out_shape=(
        jax.ShapeDtypeStruct((_N,), jnp.float32),
        jax.ShapeDtypeStruct((_N,), jnp.float32),
    ),
    in_specs=[
        pl.BlockSpec(memory_space=pltpu.VMEM),
        pl.BlockSpec(memory_space=pltpu.VMEM),
    ],
)


_NH = _N // 2
_NHP = _NH + _L
_SPT = _BT // 8


@functools.partial(
    pl.kernel,
    mesh=_mesh,
    out_type=tuple(
        jax.ShapeDtypeStruct((_BT,), jnp.float32) for _ in range(4)
    ),
    scratch_types=[
        pltpu.VMEM((_NHP,), jnp.float32),
        pltpu.VMEM((_SPT,), jnp.int32),
        pltpu.VMEM((_SPT,), jnp.float32),
    ],
    compiler_params=pltpu.CompilerParams(
        use_tc_tiling_on_sc=False, needs_layout_passes=False
    ),
)
def _gather_sc(s0_hbm, s1_hbm, idx_hbm, g00, g01, g10, g11,
               s_v, idx_v, out_v):
    wid = lax.axis_index("s") * _NC + lax.axis_index("c")
    kk = wid % 2
    half = (wid // 2) % 2
    base = (wid // 4) * _SPT
    pltpu.sync_copy(idx_hbm.at[half, pl.ds(base, _SPT)], idx_v)

    @pl.when(kk == 0)
    def _():
        pltpu.sync_copy(s0_hbm.at[pl.ds(half * _NH, _NH)],
                        s_v.at[pl.ds(0, _NH)])

    @pl.when(kk == 1)
    def _():
        pltpu.sync_copy(s1_hbm.at[pl.ds(half * _NH, _NH)],
                        s_v.at[pl.ds(0, _NH)])

    s_v[pl.ds(_NH, _L)] = jnp.zeros((_L,), jnp.float32)

    for c in range(_SPT // _L):
        iv = idx_v[pl.ds(c * _L, _L)]
        out_v[pl.ds(c * _L, _L)] = plsc.load_gather(s_v, [iv])

    outs = [[g00, g01], [g10, g11]]
    for kx in (0, 1):
        for hx in (0, 1):
            @pl.when((kk == kx) & (half == hx))
            def _(o=outs[kx][hx]):
                pltpu.sync_copy(out_v, o.at[pl.ds(base, _SPT)])


def _softplus(g, p):
    r = jnp.clip(g / p, -75.0, 75.0)
    return p * jnp.log1p(jnp.exp(r))


def _final_body(b_ref, psi_ref, k_ref, g00_ref, g01_ref,
                g10_ref, g11_ref, lam_ref, ls_ref):
    s0 = g00_ref[...] + g01_ref[...]
    s1 = g10_ref[...] + g11_ref[...]
    b0 = b_ref[0]
    b1 = b_ref[1]
    p0 = psi_ref[0]
    p1 = psi_ref[1]

    su0 = lax.slice(s0, (0,), (_B,))
    su1 = lax.slice(s1, (0,), (_B,))
    sv0 = lax.slice(s0, (_B,), (2 * _B,))
    sv1 = lax.slice(s1, (_B,), (2 * _B,))

    kk = k_ref[...]
    lam0 = _softplus(su0 + sv0 + b0, p0)
    lam1 = _softplus(su1 + sv1 + b1, p1)
    lam_ref[...] = jnp.where(kk == 0, lam0, lam1)

    ovo = 2 * _B
    ouo = ovo + _SS * _B
    for s in range(_SS):
        cv = ovo + s * _B
        cu = ouo + s * _B
        acc = (
            _softplus(su0 + lax.slice(s0, (cv,), (cv + _B,)) + b0, p0)
            + _softplus(su1 + lax.slice(s1, (cv,), (cv + _B,)) + b1, p1)
            + _softplus(sv0 + lax.slice(s0, (cu,), (cu + _B,)) + b0, p0)
            + _softplus(sv1 + lax.slice(s1, (cu,), (cu + _B,)) + b1, p1)
        )
        ls_ref[0, s] = jnp.sum(acc) * (1.0 / _SS)


_final_tc = pl.pallas_call(
    _final_body,
    out_shape=(
        jax.ShapeDtypeStruct((_B,), jnp.float32),
        jax.ShapeDtypeStruct((1, _SS), jnp.float32),
    ),
    in_specs=[pl.BlockSpec(memory_space=pltpu.SMEM)] * 2
    + [pl.BlockSpec(memory_space=pltpu.VMEM)] * 5,
    out_specs=(
        pl.BlockSpec(memory_space=pltpu.VMEM),
        pl.BlockSpec(memory_space=pltpu.SMEM),
    ),
)


def kernel(embeddings, W_omega, b_omega, psi, t, u, v, k, u_others, v_others):
    del t
    s0, s1 = _dots_tc(W_omega, embeddings.T)

    idx = jnp.concatenate([
        u.astype(jnp.int32),
        v.astype(jnp.int32),
        v_others.astype(jnp.int32).T.reshape(-1),
        u_others.astype(jnp.int32).T.reshape(-1),
    ])
    h = idx // _NH
    local = idx - h * _NH
    idx2 = jnp.where(h[None, :] == jnp.arange(2, dtype=jnp.int32)[:, None],
                     local[None, :], _NH)
    gs = _gather_sc(s0, s1, idx2)
    lam, ls = _final_tc(b_omega, psi, k.astype(jnp.int32), *gs)
    return (lam, ls.reshape(_SS))
